# Initial kernel scaffold; baseline (speedup 1.0000x reference)
import jax
import jax.numpy as jnp
from jax import lax
from jax.experimental import pallas as pl
from jax.experimental.pallas import tpu as pltpu
from jax.experimental.pallas import tpu_sc as plsc

N = 10000
NP = 10240
E = 320000
EP = 327680
D = 128
NG = 64
OUT = 323
ACC_W = 144
NTILES = 32
CHUNK = 128
CHUNKS_PER_TILE = EP // NTILES // CHUNK
ROWS_PER_TILE = NP // 16
NBLK = 8
BLK = NP // NBLK


def _pre_body(x_ref, w_ref, att_ref, h_ref, a2_ref):
    h = jnp.dot(x_ref[...], w_ref[...], preferred_element_type=jnp.float32)
    h_ref[...] = h
    a2_ref[...] = jnp.dot(h, att_ref[...], preferred_element_type=jnp.float32)


def _pre(x_pad, W, att2):
    return pl.pallas_call(
        _pre_body,
        grid=(NBLK,),
        in_specs=[
            pl.BlockSpec((BLK, D), lambda i: (i, 0)),
            pl.BlockSpec((D, D), lambda i: (0, 0)),
            pl.BlockSpec((D, 2), lambda i: (0, 0)),
        ],
        out_specs=[
            pl.BlockSpec((BLK, D), lambda i: (i, 0)),
            pl.BlockSpec((BLK, 2), lambda i: (i, 0)),
        ],
        out_shape=[
            jax.ShapeDtypeStruct((NP, D), jnp.float32),
            jax.ShapeDtypeStruct((NP, 2), jnp.float32),
        ],
    )(x_pad, W, att2)


def _edge_body(h_hbm, asrc_hbm, adst_hbm, src_hbm, dst_hbm, out_hbm,
               asrc_v, adst_v, src2d, dst2d, rows_v, out_buf, wbuf, acc, sem):
    c = lax.axis_index("c")
    s = lax.axis_index("s")
    w = c * 16 + s

    pltpu.sync_copy(asrc_hbm, asrc_v)
    pltpu.sync_copy(adst_hbm, adst_v)
    pltpu.sync_copy(src_hbm.at[pl.ds(w * CHUNKS_PER_TILE, CHUNKS_PER_TILE)], src2d)
    pltpu.sync_copy(dst_hbm.at[pl.ds(w * CHUNKS_PER_TILE, CHUNKS_PER_TILE)], dst2d)

    def _zero_row(i, carry):
        for k in range(ACC_W // 16):
            out_buf[i, pl.ds(k * 16, 16)] = jnp.zeros((16,), jnp.float32)
        return carry
    lax.fori_loop(0, CHUNK, _zero_row, 0)
    for b in range(ROWS_PER_TILE // CHUNK):
        pltpu.sync_copy(out_buf, acc.at[pl.ds(s * ROWS_PER_TILE + b * CHUNK, CHUNK)])
    plsc.subcore_barrier()

    lane0 = jnp.where(lax.iota(jnp.int32, 16) == 0,
                      jnp.float32(1.0), jnp.float32(0.0))

    def _chunk(g, carry):
        srcrow = src2d.at[g]
        dstrow = dst2d.at[g]
        pltpu.async_copy(h_hbm.at[srcrow], rows_v, sem).wait()
        for j in range(CHUNK // 16):
            s16 = src2d[g, pl.ds(j * 16, 16)]
            d16 = dst2d[g, pl.ds(j * 16, 16)]
            e = plsc.load_gather(asrc_v, [s16]) + plsc.load_gather(adst_v, [d16])
            e = jnp.maximum(e, 0.2 * e)
            wbuf[pl.ds(j * 16, 16)] = jnp.exp(e)
        def _edge_i(i, carry2):
            ws = wbuf[i]
            for k in range(D // 16):
                out_buf[i, pl.ds(k * 16, 16)] = rows_v[i, pl.ds(k * 16, 16)] * ws
            out_buf[i, pl.ds(D, 16)] = lane0 * ws
            return carry2
        lax.fori_loop(0, CHUNK, _edge_i, 0)
        pltpu.sync_copy(out_buf, acc.at[dstrow], add=True)
        return carry
    lax.fori_loop(0, CHUNKS_PER_TILE, _chunk, 0)

    plsc.subcore_barrier()
    pltpu.sync_copy(acc.at[pl.ds(s * ROWS_PER_TILE, ROWS_PER_TILE)],
                    out_hbm.at[pl.ds(c * NP + s * ROWS_PER_TILE, ROWS_PER_TILE)])


def _edge(h, a_src, a_dst, srcp, dstp):
    mesh = plsc.VectorSubcoreMesh(core_axis_name="c", subcore_axis_name="s",
                                  num_cores=2, num_subcores=16)
    fn = pl.kernel(
        _edge_body,
        out_type=jax.ShapeDtypeStruct((2 * NP, ACC_W), jnp.float32),
        mesh=mesh,
        scratch_types=[
            pltpu.VMEM((NP,), jnp.float32),
            pltpu.VMEM((NP,), jnp.float32),
            pltpu.VMEM((CHUNKS_PER_TILE, CHUNK), jnp.int32),
            pltpu.VMEM((CHUNKS_PER_TILE, CHUNK), jnp.int32),
            pltpu.VMEM((CHUNK, D), jnp.float32),
            pltpu.VMEM((CHUNK, ACC_W), jnp.float32),
            pltpu.VMEM((CHUNK,), jnp.float32),
            pltpu.VMEM_SHARED((NP, ACC_W), jnp.float32),
            pltpu.SemaphoreType.DMA,
        ],
    )
    return fn(h, a_src, a_dst, srcp, dstp)


def _post_body(acc_ref, h_ref, a2_ref, x_ref, batch_ref, bias_ref, g_ref,
               b_ref, psum_ref, pcnt_ref):
    i = pl.program_id(0)

    @pl.when(i == 0)
    def _init():
        psum_ref[...] = jnp.zeros_like(psum_ref)
        pcnt_ref[...] = jnp.zeros_like(pcnt_ref)

    acc2 = acc_ref[0] + acc_ref[1]
    rows = acc2[:, :D]
    den = jnp.sum(acc2[:, D:ACC_W], axis=1, keepdims=True)
    a2b = a2_ref[...]
    es = a2b[:, 0:1] + a2b[:, 1:2]
    es = jnp.maximum(es, 0.2 * es)
    wself = jnp.exp(es)
    rows = rows + wself * h_ref[...]
    den = den + wself
    agg = rows / (den + 1e-16) + bias_ref[...]
    h2 = agg + x_ref[...]
    mu = jnp.mean(h2, axis=1, keepdims=True)
    var = jnp.mean((h2 - mu) ** 2, axis=1, keepdims=True)
    hn = (h2 - mu) / jnp.sqrt(var + 1e-5) * g_ref[...] + b_ref[...]

    batch = batch_ref[0, 0]
    gids = lax.broadcasted_iota(jnp.int32, (NG, BLK), 0)
    bb = jnp.broadcast_to(batch.reshape(1, BLK), (NG, BLK))
    rid = i * BLK + lax.broadcasted_iota(jnp.int32, (NG, BLK), 1)
    oh = jnp.where((bb == gids) & (rid < N), jnp.float32(1.0), jnp.float32(0.0))
    psum_ref[...] += jnp.dot(oh, hn, preferred_element_type=jnp.float32)
    pcnt_ref[...] += jnp.broadcast_to(
        jnp.sum(oh, axis=1, keepdims=True), (NG, D))


def _post(acc, h, a2, x_pad, batch3, bias, gamma, beta):
    return pl.pallas_call(
        _post_body,
        grid=(NBLK,),
        in_specs=[
            pl.BlockSpec((2, BLK, ACC_W), lambda i: (0, i, 0)),
            pl.BlockSpec((BLK, D), lambda i: (i, 0)),
            pl.BlockSpec((BLK, 2), lambda i: (i, 0)),
            pl.BlockSpec((BLK, D), lambda i: (i, 0)),
            pl.BlockSpec((1, 1, BLK), lambda i: (i, 0, 0)),
            pl.BlockSpec((1, D), lambda i: (0, 0)),
            pl.BlockSpec((1, D), lambda i: (0, 0)),
            pl.BlockSpec((1, D), lambda i: (0, 0)),
        ],
        out_specs=[
            pl.BlockSpec((NG, D), lambda i: (0, 0)),
            pl.BlockSpec((NG, D), lambda i: (0, 0)),
        ],
        out_shape=[
            jax.ShapeDtypeStruct((NG, D), jnp.float32),
            jax.ShapeDtypeStruct((NG, D), jnp.float32),
        ],
    )(acc, h, a2, x_pad, batch3, bias, gamma, beta)


def _clf_body(psum_ref, pcnt_ref, w_ref, b_ref, out_ref):
    pooled = psum_ref[...] / jnp.maximum(pcnt_ref[...], 1.0)
    out_ref[...] = jnp.dot(pooled, w_ref[...],
                           preferred_element_type=jnp.float32) + b_ref[...]


def _clf(psum, pcnt, clf_W, clf_b):
    return pl.pallas_call(
        _clf_body,
        out_shape=jax.ShapeDtypeStruct((NG, OUT), jnp.float32),
    )(psum, pcnt, clf_W, clf_b)


def kernel(x, edge_index, batch, W, att_src, att_dst, bias_gat, ln_gamma,
           ln_beta, clf_W, clf_b):
    x_pad = jnp.pad(x, ((0, NP - N), (0, 0)))
    att2 = jnp.stack([att_src, att_dst], axis=1)
    h, a2 = _pre(x_pad, W, att2)
    a_src = a2[:, 0]
    a_dst = a2[:, 1]
    pad = jnp.full((EP - E,), N, dtype=edge_index.dtype)
    srcp = jnp.concatenate([edge_index[0], pad]).reshape(EP // CHUNK, CHUNK)
    dstp = jnp.concatenate([edge_index[1], pad]).reshape(EP // CHUNK, CHUNK)
    acc = _edge(h, a_src, a_dst, srcp, dstp).reshape(2, NP, ACC_W)
    batch3 = jnp.pad(batch, (0, NP - N), constant_values=NG).reshape(NBLK, 1, BLK)
    psum, pcnt = _post(acc, h, a2, x_pad, batch3, bias_gat.reshape(1, D),
                       ln_gamma.reshape(1, D), ln_beta.reshape(1, D))
    return _clf(psum, pcnt, clf_W, clf_b.reshape(1, OUT))


# speedup vs baseline: 5.4563x; 5.4563x over previous
import jax
import jax.numpy as jnp
from jax import lax
from jax.experimental import pallas as pl
from jax.experimental.pallas import tpu as pltpu
from jax.experimental.pallas import tpu_sc as plsc

N = 10000
NP = 10240
E = 320000
EP = 327680
D = 128
NG = 64
OUT = 323
NTILES = 32
CHUNK = 128
NH = NP // 2
NHT = NH + CHUNK
CHUNKS_PER_TILE = EP // 16 // CHUNK
ROWS_PER_TILE = NHT // 16
NBLK = 8
BLK = NP // NBLK


def _pre_body(x_ref, w_ref, att_ref, h_ref, a2_ref):
    h = jnp.dot(x_ref[...], w_ref[...], preferred_element_type=jnp.float32)
    h_ref[...] = h
    a2_ref[...] = jnp.dot(h, att_ref[...], preferred_element_type=jnp.float32)


def _pre(x_pad, W, att2):
    return pl.pallas_call(
        _pre_body,
        grid=(NBLK,),
        in_specs=[
            pl.BlockSpec((BLK, D), lambda i: (i, 0)),
            pl.BlockSpec((D, D), lambda i: (0, 0)),
            pl.BlockSpec((D, 2), lambda i: (0, 0)),
        ],
        out_specs=[
            pl.BlockSpec((BLK, D), lambda i: (i, 0)),
            pl.BlockSpec((BLK, 2), lambda i: (i, 0)),
        ],
        out_shape=[
            jax.ShapeDtypeStruct((NP, D), jnp.float32),
            jax.ShapeDtypeStruct((NP, 2), jnp.float32),
        ],
    )(x_pad, W, att2)


def _make_edge_body(ci):
    base_node = ci * NH

    def _edge_body(h_hbm, asrc_hbm, adst_hbm, epk_hbm, out_hbm,
                   den_hbm, asrc_v, adst_v, epk2d, rows_v, out_buf,
                   wbuf, sidx, dloc, den_v, acc, sem):
        s = lax.axis_index("s")

        pltpu.sync_copy(asrc_hbm, asrc_v)
        pltpu.sync_copy(adst_hbm, adst_v)
        pltpu.sync_copy(epk_hbm.at[pl.ds(s * CHUNKS_PER_TILE, CHUNKS_PER_TILE)],
                        epk2d)

        zero16 = jnp.zeros((16,), jnp.float32)

        def _zero_row(i, carry):
            for k in range(D // 16):
                out_buf[i, pl.ds(k * 16, 16)] = zero16
            return carry
        lax.fori_loop(0, CHUNK, _zero_row, 0)

        def _zero_den(i, carry):
            den_v[pl.ds(i * 16, 16)] = zero16
            return carry
        lax.fori_loop(0, NHT // 16, _zero_den, 0)

        pltpu.sync_copy(out_buf, acc.at[pl.ds(s * ROWS_PER_TILE, CHUNK)])
        pltpu.sync_copy(out_buf,
                        acc.at[pl.ds(s * ROWS_PER_TILE + CHUNK, CHUNK)])
        pltpu.sync_copy(out_buf.at[pl.ds(0, ROWS_PER_TILE - 2 * CHUNK)],
                        acc.at[pl.ds(s * ROWS_PER_TILE + 2 * CHUNK,
                                     ROWS_PER_TILE - 2 * CHUNK)])
        plsc.subcore_barrier()

        lanes = lax.iota(jnp.int32, 16)

        def _chunk(g, carry):
            for j in range(CHUNK // 16):
                pk = epk2d[g, pl.ds(j * 16, 16)]
                s16 = lax.shift_right_logical(pk, 14)
                d16 = pk & ((1 << 14) - 1)
                sidx[pl.ds(j * 16, 16)] = s16
                dl = d16 - base_node
                dl = jnp.where((dl >= 0) & (dl < NH), dl, NH)
                dloc[pl.ds(j * 16, 16)] = dl
                e = (plsc.load_gather(asrc_v, [s16])
                     + plsc.load_gather(adst_v, [d16]))
                e = jnp.maximum(e, 0.2 * e)
                w16 = jnp.exp(e)
                wbuf[pl.ds(j * 16, 16)] = w16
                for l in range(16):
                    plsc.addupdate_scatter(den_v, [dl], w16, mask=lanes == l)
            pltpu.async_copy(h_hbm.at[sidx], rows_v, sem).wait()
            def _edge_grp(j, carry2):
                w16 = wbuf[pl.ds(j * 16, 16)]
                base = j * 16
                for l in range(16):
                    ws = w16[l]
                    for k in range(D // 16):
                        out_buf[base + l, pl.ds(k * 16, 16)] = (
                            rows_v[base + l, pl.ds(k * 16, 16)] * ws)
                return carry2
            lax.fori_loop(0, CHUNK // 16, _edge_grp, 0)
            pltpu.sync_copy(out_buf, acc.at[dloc], add=True)
            return carry
        lax.fori_loop(0, CHUNKS_PER_TILE, _chunk, 0)

        plsc.subcore_barrier()
        pltpu.sync_copy(acc.at[pl.ds(s * ROWS_PER_TILE, ROWS_PER_TILE)],
                        out_hbm.at[pl.ds(s * ROWS_PER_TILE, ROWS_PER_TILE)])
        pltpu.sync_copy(den_v, den_hbm.at[s])

    return _edge_body


def _edge_half(ci, h, a_src, a_dst, epk):
    mesh = plsc.VectorSubcoreMesh(core_axis_name="c", subcore_axis_name="s",
                                  num_cores=1, num_subcores=16)
    fn = pl.kernel(
        _make_edge_body(ci),
        out_type=[
            jax.ShapeDtypeStruct((NHT, D), jnp.float32),
            jax.ShapeDtypeStruct((16, NHT), jnp.float32),
        ],
        mesh=mesh,
        scratch_types=[
            pltpu.VMEM((NP,), jnp.float32),
            pltpu.VMEM((NP,), jnp.float32),
            pltpu.VMEM((CHUNKS_PER_TILE, CHUNK), jnp.int32),
            pltpu.VMEM((CHUNK, D), jnp.float32),
            pltpu.VMEM((CHUNK, D), jnp.float32),
            pltpu.VMEM((CHUNK,), jnp.float32),
            pltpu.VMEM((CHUNK,), jnp.int32),
            pltpu.VMEM((CHUNK,), jnp.int32),
            pltpu.VMEM((NHT,), jnp.float32),
            pltpu.VMEM_SHARED((NHT, D), jnp.float32),
            pltpu.SemaphoreType.DMA,
        ],
        compiler_params=pltpu.CompilerParams(needs_layout_passes=False),
    )
    return fn(h, a_src, a_dst, epk)


def _post_body(acc_ref, den_ref, h_ref, a2_ref, x_ref, batch_ref, bias_ref,
               g_ref, b_ref, psum_ref, pcnt_ref):
    i = pl.program_id(0)

    @pl.when(i == 0)
    def _init():
        psum_ref[...] = jnp.zeros_like(psum_ref)
        pcnt_ref[...] = jnp.zeros_like(pcnt_ref)

    rows = acc_ref[0]
    dsum = jnp.sum(den_ref[0], axis=0, keepdims=True)
    den = lax.transpose(dsum, (1, 0))
    a2b = a2_ref[...]
    es = a2b[:, 0:1] + a2b[:, 1:2]
    es = jnp.maximum(es, 0.2 * es)
    wself = jnp.exp(es)
    rows = rows + wself * h_ref[...]
    den = den + wself
    agg = rows / (den + 1e-16) + bias_ref[...]
    h2 = agg + x_ref[...]
    mu = jnp.mean(h2, axis=1, keepdims=True)
    var = jnp.mean((h2 - mu) ** 2, axis=1, keepdims=True)
    hn = (h2 - mu) / jnp.sqrt(var + 1e-5) * g_ref[...] + b_ref[...]

    batch = batch_ref[0, 0]
    gids = lax.broadcasted_iota(jnp.int32, (NG, BLK), 0)
    bb = jnp.broadcast_to(batch.reshape(1, BLK), (NG, BLK))
    rid = i * BLK + lax.broadcasted_iota(jnp.int32, (NG, BLK), 1)
    oh = jnp.where((bb == gids) & (rid < N), jnp.float32(1.0), jnp.float32(0.0))
    psum_ref[...] += jnp.dot(oh, hn, preferred_element_type=jnp.float32)
    pcnt_ref[...] += jnp.broadcast_to(
        jnp.sum(oh, axis=1, keepdims=True), (NG, D))


def _post(acc, den, h, a2, x_pad, batch3, bias, gamma, beta):
    return pl.pallas_call(
        _post_body,
        grid=(NBLK,),
        in_specs=[
            pl.BlockSpec((1, BLK, D), lambda i: (i // 4, i % 4, 0)),
            pl.BlockSpec((1, 16, BLK), lambda i: (i // 4, 0, i % 4)),
            pl.BlockSpec((BLK, D), lambda i: (i, 0)),
            pl.BlockSpec((BLK, 2), lambda i: (i, 0)),
            pl.BlockSpec((BLK, D), lambda i: (i, 0)),
            pl.BlockSpec((1, 1, BLK), lambda i: (i, 0, 0)),
            pl.BlockSpec((1, D), lambda i: (0, 0)),
            pl.BlockSpec((1, D), lambda i: (0, 0)),
            pl.BlockSpec((1, D), lambda i: (0, 0)),
        ],
        out_specs=[
            pl.BlockSpec((NG, D), lambda i: (0, 0)),
            pl.BlockSpec((NG, D), lambda i: (0, 0)),
        ],
        out_shape=[
            jax.ShapeDtypeStruct((NG, D), jnp.float32),
            jax.ShapeDtypeStruct((NG, D), jnp.float32),
        ],
    )(acc, den, h, a2, x_pad, batch3, bias, gamma, beta)


def _clf_body(psum_ref, pcnt_ref, w_ref, b_ref, out_ref):
    pooled = psum_ref[...] / jnp.maximum(pcnt_ref[...], 1.0)
    out_ref[...] = jnp.dot(pooled, w_ref[...],
                           preferred_element_type=jnp.float32) + b_ref[...]


def _clf(psum, pcnt, clf_W, clf_b):
    return pl.pallas_call(
        _clf_body,
        out_shape=jax.ShapeDtypeStruct((NG, OUT), jnp.float32),
    )(psum, pcnt, clf_W, clf_b)


def kernel(x, edge_index, batch, W, att_src, att_dst, bias_gat, ln_gamma,
           ln_beta, clf_W, clf_b):
    x_pad = jnp.pad(x, ((0, NP - N), (0, 0)))
    att2 = jnp.stack([att_src, att_dst], axis=1)
    h, a2 = _pre(x_pad, W, att2)
    a_src = a2[:, 0]
    a_dst = a2[:, 1]
    pad = jnp.full((EP - E,), N, dtype=edge_index.dtype)
    srcp = jnp.concatenate([edge_index[0], pad])
    dstp = jnp.concatenate([edge_index[1], pad])
    epk = ((srcp << 14) | dstp).reshape(EP // CHUNK, CHUNK)
    accf0, den0 = _edge_half(0, h, a_src, a_dst, epk)
    accf1, den1 = _edge_half(1, h, a_src, a_dst, epk)
    acc = jnp.stack([accf0, accf1])
    den = jnp.stack([den0, den1])
    batch3 = jnp.pad(batch, (0, NP - N), constant_values=NG).reshape(NBLK, 1, BLK)
    psum, pcnt = _post(acc, den, h, a2, x_pad, batch3, bias_gat.reshape(1, D),
                       ln_gamma.reshape(1, D), ln_beta.reshape(1, D))
    return _clf(psum, pcnt, clf_W, clf_b.reshape(1, OUT))


# speedup vs baseline: 7.5922x; 1.3914x over previous
import jax
import jax.numpy as jnp
from jax import lax
from jax.experimental import pallas as pl
from jax.experimental.pallas import tpu as pltpu
from jax.experimental.pallas import tpu_sc as plsc

N = 10000
NP = 10240
E = 320000
EP = 327680
D = 128
NG = 64
OUT = 323
NTILES = 32
CHUNK = 128
NH = NP // 2
NHT = NH + CHUNK
CHUNKS_PER_TILE = EP // 16 // CHUNK
ROWS_PER_TILE = NHT // 16
NBLK = 8
BLK = NP // NBLK


def _pre_body(x_ref, w_ref, att_ref, h_ref, a2_ref):
    h = jnp.dot(x_ref[...], w_ref[...], preferred_element_type=jnp.float32)
    h_ref[...] = h
    a2_ref[...] = jnp.dot(h, att_ref[...], preferred_element_type=jnp.float32)


def _pre(x_pad, W, att2):
    return pl.pallas_call(
        _pre_body,
        grid=(NBLK,),
        in_specs=[
            pl.BlockSpec((BLK, D), lambda i: (i, 0)),
            pl.BlockSpec((D, D), lambda i: (0, 0)),
            pl.BlockSpec((D, 2), lambda i: (0, 0)),
        ],
        out_specs=[
            pl.BlockSpec((BLK, D), lambda i: (i, 0)),
            pl.BlockSpec((BLK, 2), lambda i: (i, 0)),
        ],
        out_shape=[
            jax.ShapeDtypeStruct((NP, D), jnp.float32),
            jax.ShapeDtypeStruct((NP, 2), jnp.float32),
        ],
    )(x_pad, W, att2)


def _make_edge_body(ci):
    base_node = ci * NH

    def _edge_body(h_hbm, asrc_hbm, adst_hbm, epk_hbm, out_hbm,
                   den_hbm, asrc_v, adst_v, epk2d, rows_a, rows_b,
                   wbuf_a, wbuf_b, sidx_a, sidx_b, dloc_a, dloc_b,
                   den_v, acc, gsem_a, gsem_b, ssem_a, ssem_b):
        s = lax.axis_index("s")

        pltpu.sync_copy(asrc_hbm, asrc_v)
        pltpu.sync_copy(adst_hbm, adst_v)
        pltpu.sync_copy(epk_hbm.at[pl.ds(s * CHUNKS_PER_TILE, CHUNKS_PER_TILE)],
                        epk2d)

        zero16 = jnp.zeros((16,), jnp.float32)

        def _zero_row(i, carry):
            for k in range(D // 16):
                rows_a[i, pl.ds(k * 16, 16)] = zero16
            return carry
        lax.fori_loop(0, CHUNK, _zero_row, 0)

        def _zero_den(i, carry):
            den_v[pl.ds(i * 16, 16)] = zero16
            return carry
        lax.fori_loop(0, NHT // 16, _zero_den, 0)

        pltpu.sync_copy(rows_a, acc.at[pl.ds(s * ROWS_PER_TILE, CHUNK)])
        pltpu.sync_copy(rows_a,
                        acc.at[pl.ds(s * ROWS_PER_TILE + CHUNK, CHUNK)])
        pltpu.sync_copy(rows_a.at[pl.ds(0, ROWS_PER_TILE - 2 * CHUNK)],
                        acc.at[pl.ds(s * ROWS_PER_TILE + 2 * CHUNK,
                                     ROWS_PER_TILE - 2 * CHUNK)])
        plsc.subcore_barrier()

        lanes = lax.iota(jnp.int32, 16)

        def _compute(g, sidx_b, dloc_b, wbuf_b):
            for j in range(CHUNK // 16):
                pk = epk2d[g, pl.ds(j * 16, 16)]
                s16 = lax.shift_right_logical(pk, 14)
                d16 = pk & ((1 << 14) - 1)
                sidx_b[pl.ds(j * 16, 16)] = s16
                dl = d16 - base_node
                dl = jnp.where((dl >= 0) & (dl < NH), dl, NH)
                dloc_b[pl.ds(j * 16, 16)] = dl
                e = (plsc.load_gather(asrc_v, [s16])
                     + plsc.load_gather(adst_v, [d16]))
                e = jnp.maximum(e, 0.2 * e)
                w16 = jnp.exp(e)
                wbuf_b[pl.ds(j * 16, 16)] = w16
                for l in range(16):
                    plsc.addupdate_scatter(den_v, [dl], w16, mask=lanes == l)

        def _scale(rows_x, wbuf_x):
            def _grp(j, carry2):
                w16 = wbuf_x[pl.ds(j * 16, 16)]
                base = j * 16
                for l in range(16):
                    ws = w16[l]
                    for k in range(D // 16):
                        rows_x[base + l, pl.ds(k * 16, 16)] = (
                            rows_x[base + l, pl.ds(k * 16, 16)] * ws)
                return carry2
            lax.fori_loop(0, CHUNK // 16, _grp, 0)

        sets = ((sidx_a, dloc_a, wbuf_a, rows_a, gsem_a, ssem_a),
                (sidx_b, dloc_b, wbuf_b, rows_b, gsem_b, ssem_b))

        _compute(0, sidx_a, dloc_a, wbuf_a)
        pltpu.async_copy(h_hbm.at[sidx_a], rows_a, gsem_a)

        def _pair(t, carry):
            for half in range(2):
                sx, dx, wx, rx, gx, ssx = sets[1 - half]
                sy, dy, wy, ry, gy, ssy = sets[half]
                nxt = 2 * t + 1 + half
                cur = 2 * t + half

                @pl.when(nxt < CHUNKS_PER_TILE)
                def _fill():
                    @pl.when(nxt >= 2)
                    def _drain_prev_scatter():
                        pltpu.make_async_copy(rx, acc.at[dx], ssx).wait()
                    _compute(nxt, sx, dx, wx)
                    pltpu.async_copy(h_hbm.at[sx], rx, gx)

                pltpu.make_async_copy(h_hbm.at[sy], ry, gy).wait()
                _scale(ry, wy)
                pltpu.async_copy(ry, acc.at[dy], ssy, add=True)
            return carry
        lax.fori_loop(0, CHUNKS_PER_TILE // 2, _pair, 0)

        pltpu.make_async_copy(rows_a, acc.at[dloc_a], ssem_a).wait()
        pltpu.make_async_copy(rows_b, acc.at[dloc_b], ssem_b).wait()

        plsc.subcore_barrier()
        pltpu.sync_copy(acc.at[pl.ds(s * ROWS_PER_TILE, ROWS_PER_TILE)],
                        out_hbm.at[pl.ds(s * ROWS_PER_TILE, ROWS_PER_TILE)])
        pltpu.sync_copy(den_v, den_hbm.at[s])

    return _edge_body


def _edge_half(ci, h, a_src, a_dst, epk):
    mesh = plsc.VectorSubcoreMesh(core_axis_name="c", subcore_axis_name="s",
                                  num_cores=1, num_subcores=16)
    fn = pl.kernel(
        _make_edge_body(ci),
        out_type=[
            jax.ShapeDtypeStruct((NHT, D), jnp.float32),
            jax.ShapeDtypeStruct((16, NHT), jnp.float32),
        ],
        mesh=mesh,
        scratch_types=[
            pltpu.VMEM((NP,), jnp.float32),
            pltpu.VMEM((NP,), jnp.float32),
            pltpu.VMEM((CHUNKS_PER_TILE, CHUNK), jnp.int32),
            pltpu.VMEM((CHUNK, D), jnp.float32),
            pltpu.VMEM((CHUNK, D), jnp.float32),
            pltpu.VMEM((CHUNK,), jnp.float32),
            pltpu.VMEM((CHUNK,), jnp.float32),
            pltpu.VMEM((CHUNK,), jnp.int32),
            pltpu.VMEM((CHUNK,), jnp.int32),
            pltpu.VMEM((CHUNK,), jnp.int32),
            pltpu.VMEM((CHUNK,), jnp.int32),
            pltpu.VMEM((NHT,), jnp.float32),
            pltpu.VMEM_SHARED((NHT, D), jnp.float32),
            pltpu.SemaphoreType.DMA,
            pltpu.SemaphoreType.DMA,
            pltpu.SemaphoreType.DMA,
            pltpu.SemaphoreType.DMA,
        ],
        compiler_params=pltpu.CompilerParams(needs_layout_passes=False),
    )
    return fn(h, a_src, a_dst, epk)


def _post_body(acc_ref, den_ref, h_ref, a2_ref, x_ref, batch_ref, bias_ref,
               g_ref, b_ref, psum_ref, pcnt_ref):
    i = pl.program_id(0)

    @pl.when(i == 0)
    def _init():
        psum_ref[...] = jnp.zeros_like(psum_ref)
        pcnt_ref[...] = jnp.zeros_like(pcnt_ref)

    rows = acc_ref[0]
    dsum = jnp.sum(den_ref[0], axis=0, keepdims=True)
    den = lax.transpose(dsum, (1, 0))
    a2b = a2_ref[...]
    es = a2b[:, 0:1] + a2b[:, 1:2]
    es = jnp.maximum(es, 0.2 * es)
    wself = jnp.exp(es)
    rows = rows + wself * h_ref[...]
    den = den + wself
    agg = rows / (den + 1e-16) + bias_ref[...]
    h2 = agg + x_ref[...]
    mu = jnp.mean(h2, axis=1, keepdims=True)
    var = jnp.mean((h2 - mu) ** 2, axis=1, keepdims=True)
    hn = (h2 - mu) / jnp.sqrt(var + 1e-5) * g_ref[...] + b_ref[...]

    batch = batch_ref[0, 0]
    gids = lax.broadcasted_iota(jnp.int32, (NG, BLK), 0)
    bb = jnp.broadcast_to(batch.reshape(1, BLK), (NG, BLK))
    rid = i * BLK + lax.broadcasted_iota(jnp.int32, (NG, BLK), 1)
    oh = jnp.where((bb == gids) & (rid < N), jnp.float32(1.0), jnp.float32(0.0))
    psum_ref[...] += jnp.dot(oh, hn, preferred_element_type=jnp.float32)
    pcnt_ref[...] += jnp.broadcast_to(
        jnp.sum(oh, axis=1, keepdims=True), (NG, D))


def _post(acc, den, h, a2, x_pad, batch3, bias, gamma, beta):
    return pl.pallas_call(
        _post_body,
        grid=(NBLK,),
        in_specs=[
            pl.BlockSpec((1, BLK, D), lambda i: (i // 4, i % 4, 0)),
            pl.BlockSpec((1, 16, BLK), lambda i: (i // 4, 0, i % 4)),
            pl.BlockSpec((BLK, D), lambda i: (i, 0)),
            pl.BlockSpec((BLK, 2), lambda i: (i, 0)),
            pl.BlockSpec((BLK, D), lambda i: (i, 0)),
            pl.BlockSpec((1, 1, BLK), lambda i: (i, 0, 0)),
            pl.BlockSpec((1, D), lambda i: (0, 0)),
            pl.BlockSpec((1, D), lambda i: (0, 0)),
            pl.BlockSpec((1, D), lambda i: (0, 0)),
        ],
        out_specs=[
            pl.BlockSpec((NG, D), lambda i: (0, 0)),
            pl.BlockSpec((NG, D), lambda i: (0, 0)),
        ],
        out_shape=[
            jax.ShapeDtypeStruct((NG, D), jnp.float32),
            jax.ShapeDtypeStruct((NG, D), jnp.float32),
        ],
    )(acc, den, h, a2, x_pad, batch3, bias, gamma, beta)


def _clf_body(psum_ref, pcnt_ref, w_ref, b_ref, out_ref):
    pooled = psum_ref[...] / jnp.maximum(pcnt_ref[...], 1.0)
    out_ref[...] = jnp.dot(pooled, w_ref[...],
                           preferred_element_type=jnp.float32) + b_ref[...]


def _clf(psum, pcnt, clf_W, clf_b):
    return pl.pallas_call(
        _clf_body,
        out_shape=jax.ShapeDtypeStruct((NG, OUT), jnp.float32),
    )(psum, pcnt, clf_W, clf_b)


def kernel(x, edge_index, batch, W, att_src, att_dst, bias_gat, ln_gamma,
           ln_beta, clf_W, clf_b):
    x_pad = jnp.pad(x, ((0, NP - N), (0, 0)))
    att2 = jnp.stack([att_src, att_dst], axis=1)
    h, a2 = _pre(x_pad, W, att2)
    a_src = a2[:, 0]
    a_dst = a2[:, 1]
    pad = jnp.full((EP - E,), N, dtype=edge_index.dtype)
    srcp = jnp.concatenate([edge_index[0], pad])
    dstp = jnp.concatenate([edge_index[1], pad])
    epk = ((srcp << 14) | dstp).reshape(EP // CHUNK, CHUNK)
    accf0, den0 = _edge_half(0, h, a_src, a_dst, epk)
    accf1, den1 = _edge_half(1, h, a_src, a_dst, epk)
    acc = jnp.stack([accf0, accf1])
    den = jnp.stack([den0, den1])
    batch3 = jnp.pad(batch, (0, NP - N), constant_values=NG).reshape(NBLK, 1, BLK)
    psum, pcnt = _post(acc, den, h, a2, x_pad, batch3, bias_gat.reshape(1, D),
                       ln_gamma.reshape(1, D), ln_beta.reshape(1, D))
    return _clf(psum, pcnt, clf_W, clf_b.reshape(1, OUT))


# speedup vs baseline: 10.7788x; 1.4197x over previous
import jax
import jax.numpy as jnp
from jax import lax
from jax.experimental import pallas as pl
from jax.experimental.pallas import tpu as pltpu
from jax.experimental.pallas import tpu_sc as plsc

N = 10000
NP = 10240
E = 320000
EP = 327680
D = 128
NG = 64
OUT = 323
NTILES = 32
CHUNK = 128
NH = NP // 2
NHT = NH + CHUNK
CHUNKS_PER_TILE = EP // 16 // CHUNK
ROWS_PER_TILE = NHT // 16
NBLK = 8
BLK = NP // NBLK


def _pre_body(x_ref, w_ref, att_ref, h_ref, a2_ref):
    h = jnp.dot(x_ref[...], w_ref[...], preferred_element_type=jnp.float32)
    h_ref[...] = h
    a2_ref[...] = jnp.dot(h, att_ref[...], preferred_element_type=jnp.float32)


def _pre(x_pad, W, att2):
    return pl.pallas_call(
        _pre_body,
        grid=(NBLK,),
        in_specs=[
            pl.BlockSpec((BLK, D), lambda i: (i, 0)),
            pl.BlockSpec((D, D), lambda i: (0, 0)),
            pl.BlockSpec((D, 2), lambda i: (0, 0)),
        ],
        out_specs=[
            pl.BlockSpec((BLK, D), lambda i: (i, 0)),
            pl.BlockSpec((BLK, 2), lambda i: (i, 0)),
        ],
        out_shape=[
            jax.ShapeDtypeStruct((NP, D), jnp.float32),
            jax.ShapeDtypeStruct((NP, 2), jnp.float32),
        ],
    )(x_pad, W, att2)


def _make_edge_body():
    def _edge_body(h_hbm, asrc_hbm, adst_hbm, epk_hbm, out_hbm,
                   den_hbm, asrc_v, adst_v, epk2d, rows_a, rows_b,
                   wbuf_a, wbuf_b, sidx_a, sidx_b, dloc_a, dloc_b,
                   den_v, acc, gsem_a, gsem_b, ssem_a, ssem_b):
        c = lax.axis_index("c")
        s = lax.axis_index("s")
        base_node = c * NH
        out_base = c * NHT

        pltpu.sync_copy(asrc_hbm, asrc_v)
        pltpu.sync_copy(adst_hbm, adst_v)
        pltpu.sync_copy(epk_hbm.at[pl.ds(s * CHUNKS_PER_TILE, CHUNKS_PER_TILE)],
                        epk2d)

        zero16 = jnp.zeros((16,), jnp.float32)

        def _zero_row(i, carry):
            for k in range(D // 16):
                rows_a[i, pl.ds(k * 16, 16)] = zero16
            return carry
        lax.fori_loop(0, CHUNK, _zero_row, 0)

        def _zero_den(i, carry):
            den_v[pl.ds(i * 16, 16)] = zero16
            return carry
        lax.fori_loop(0, NHT // 16, _zero_den, 0)

        pltpu.sync_copy(rows_a, acc.at[pl.ds(s * ROWS_PER_TILE, CHUNK)])
        pltpu.sync_copy(rows_a,
                        acc.at[pl.ds(s * ROWS_PER_TILE + CHUNK, CHUNK)])
        pltpu.sync_copy(rows_a.at[pl.ds(0, ROWS_PER_TILE - 2 * CHUNK)],
                        acc.at[pl.ds(s * ROWS_PER_TILE + 2 * CHUNK,
                                     ROWS_PER_TILE - 2 * CHUNK)])
        plsc.subcore_barrier()

        lanes = lax.iota(jnp.int32, 16)

        def _compute(g, sidx_b, dloc_b, wbuf_b):
            for j in range(CHUNK // 16):
                pk = epk2d[g, pl.ds(j * 16, 16)]
                s16 = lax.shift_right_logical(pk, 14)
                d16 = pk & ((1 << 14) - 1)
                sidx_b[pl.ds(j * 16, 16)] = s16
                dl = d16 - base_node
                dl = jnp.where((dl >= 0) & (dl < NH), dl, NH)
                dloc_b[pl.ds(j * 16, 16)] = dl
                e = (plsc.load_gather(asrc_v, [s16])
                     + plsc.load_gather(adst_v, [d16]))
                e = jnp.maximum(e, 0.2 * e)
                w16 = jnp.exp(e)
                wbuf_b[pl.ds(j * 16, 16)] = w16
                for l in range(16):
                    plsc.addupdate_scatter(den_v, [dl], w16, mask=lanes == l)

        def _scale(rows_x, wbuf_x):
            def _grp(j, carry2):
                w16 = wbuf_x[pl.ds(j * 16, 16)]
                base = j * 16
                for l in range(16):
                    ws = w16[l]
                    for k in range(D // 16):
                        rows_x[base + l, pl.ds(k * 16, 16)] = (
                            rows_x[base + l, pl.ds(k * 16, 16)] * ws)
                return carry2
            lax.fori_loop(0, CHUNK // 16, _grp, 0)

        sets = ((sidx_a, dloc_a, wbuf_a, rows_a, gsem_a, ssem_a),
                (sidx_b, dloc_b, wbuf_b, rows_b, gsem_b, ssem_b))

        _compute(0, sidx_a, dloc_a, wbuf_a)
        pltpu.async_copy(h_hbm.at[sidx_a], rows_a, gsem_a)

        def _pair(t, carry):
            for half in range(2):
                sx, dx, wx, rx, gx, ssx = sets[1 - half]
                sy, dy, wy, ry, gy, ssy = sets[half]
                nxt = 2 * t + 1 + half
                cur = 2 * t + half

                @pl.when(nxt < CHUNKS_PER_TILE)
                def _fill():
                    @pl.when(nxt >= 2)
                    def _drain_prev_scatter():
                        pltpu.make_async_copy(rx, acc.at[dx], ssx).wait()
                    _compute(nxt, sx, dx, wx)
                    pltpu.async_copy(h_hbm.at[sx], rx, gx)

                pltpu.make_async_copy(h_hbm.at[sy], ry, gy).wait()
                _scale(ry, wy)
                pltpu.async_copy(ry, acc.at[dy], ssy, add=True)
            return carry
        lax.fori_loop(0, CHUNKS_PER_TILE // 2, _pair, 0)

        pltpu.make_async_copy(rows_a, acc.at[dloc_a], ssem_a).wait()
        pltpu.make_async_copy(rows_b, acc.at[dloc_b], ssem_b).wait()

        plsc.subcore_barrier()
        pltpu.sync_copy(acc.at[pl.ds(s * ROWS_PER_TILE, ROWS_PER_TILE)],
                        out_hbm.at[pl.ds(out_base + s * ROWS_PER_TILE,
                                         ROWS_PER_TILE)])
        pltpu.sync_copy(den_v, den_hbm.at[c * 16 + s])

    return _edge_body


def _edge_all(h, a_src, a_dst, epk):
    mesh = plsc.VectorSubcoreMesh(core_axis_name="c", subcore_axis_name="s",
                                  num_cores=2, num_subcores=16)
    fn = pl.kernel(
        _make_edge_body(),
        out_type=[
            jax.ShapeDtypeStruct((2 * NHT, D), jnp.float32),
            jax.ShapeDtypeStruct((NTILES, NHT), jnp.float32),
        ],
        mesh=mesh,
        scratch_types=[
            pltpu.VMEM((NP,), jnp.float32),
            pltpu.VMEM((NP,), jnp.float32),
            pltpu.VMEM((CHUNKS_PER_TILE, CHUNK), jnp.int32),
            pltpu.VMEM((CHUNK, D), jnp.float32),
            pltpu.VMEM((CHUNK, D), jnp.float32),
            pltpu.VMEM((CHUNK,), jnp.float32),
            pltpu.VMEM((CHUNK,), jnp.float32),
            pltpu.VMEM((CHUNK,), jnp.int32),
            pltpu.VMEM((CHUNK,), jnp.int32),
            pltpu.VMEM((CHUNK,), jnp.int32),
            pltpu.VMEM((CHUNK,), jnp.int32),
            pltpu.VMEM((NHT,), jnp.float32),
            pltpu.VMEM_SHARED((NHT, D), jnp.float32),
            pltpu.SemaphoreType.DMA,
            pltpu.SemaphoreType.DMA,
            pltpu.SemaphoreType.DMA,
            pltpu.SemaphoreType.DMA,
        ],
        compiler_params=pltpu.CompilerParams(needs_layout_passes=False),
    )
    return fn(h, a_src, a_dst, epk)


def _post_body(acc_ref, den_ref, h_ref, a2_ref, x_ref, batch_ref, bias_ref,
               g_ref, b_ref, psum_ref, pcnt_ref):
    i = pl.program_id(0)

    @pl.when(i == 0)
    def _init():
        psum_ref[...] = jnp.zeros_like(psum_ref)
        pcnt_ref[...] = jnp.zeros_like(pcnt_ref)

    rows = acc_ref[0]
    dsum = jnp.sum(den_ref[0], axis=0, keepdims=True)
    den = lax.transpose(dsum, (1, 0))
    a2b = a2_ref[...]
    es = a2b[:, 0:1] + a2b[:, 1:2]
    es = jnp.maximum(es, 0.2 * es)
    wself = jnp.exp(es)
    rows = rows + wself * h_ref[...]
    den = den + wself
    agg = rows / (den + 1e-16) + bias_ref[...]
    h2 = agg + x_ref[...]
    mu = jnp.mean(h2, axis=1, keepdims=True)
    var = jnp.mean((h2 - mu) ** 2, axis=1, keepdims=True)
    hn = (h2 - mu) / jnp.sqrt(var + 1e-5) * g_ref[...] + b_ref[...]

    batch = batch_ref[0, 0]
    gids = lax.broadcasted_iota(jnp.int32, (NG, BLK), 0)
    bb = jnp.broadcast_to(batch.reshape(1, BLK), (NG, BLK))
    rid = i * BLK + lax.broadcasted_iota(jnp.int32, (NG, BLK), 1)
    oh = jnp.where((bb == gids) & (rid < N), jnp.float32(1.0), jnp.float32(0.0))
    psum_ref[...] += jnp.dot(oh, hn, preferred_element_type=jnp.float32)
    pcnt_ref[...] += jnp.broadcast_to(
        jnp.sum(oh, axis=1, keepdims=True), (NG, D))


def _post(acc, den, h, a2, x_pad, batch3, bias, gamma, beta):
    return pl.pallas_call(
        _post_body,
        grid=(NBLK,),
        in_specs=[
            pl.BlockSpec((1, BLK, D), lambda i: (i // 4, i % 4, 0)),
            pl.BlockSpec((1, 16, BLK), lambda i: (i // 4, 0, i % 4)),
            pl.BlockSpec((BLK, D), lambda i: (i, 0)),
            pl.BlockSpec((BLK, 2), lambda i: (i, 0)),
            pl.BlockSpec((BLK, D), lambda i: (i, 0)),
            pl.BlockSpec((1, 1, BLK), lambda i: (i, 0, 0)),
            pl.BlockSpec((1, D), lambda i: (0, 0)),
            pl.BlockSpec((1, D), lambda i: (0, 0)),
            pl.BlockSpec((1, D), lambda i: (0, 0)),
        ],
        out_specs=[
            pl.BlockSpec((NG, D), lambda i: (0, 0)),
            pl.BlockSpec((NG, D), lambda i: (0, 0)),
        ],
        out_shape=[
            jax.ShapeDtypeStruct((NG, D), jnp.float32),
            jax.ShapeDtypeStruct((NG, D), jnp.float32),
        ],
    )(acc, den, h, a2, x_pad, batch3, bias, gamma, beta)


def _clf_body(psum_ref, pcnt_ref, w_ref, b_ref, out_ref):
    pooled = psum_ref[...] / jnp.maximum(pcnt_ref[...], 1.0)
    out_ref[...] = jnp.dot(pooled, w_ref[...],
                           preferred_element_type=jnp.float32) + b_ref[...]


def _clf(psum, pcnt, clf_W, clf_b):
    return pl.pallas_call(
        _clf_body,
        out_shape=jax.ShapeDtypeStruct((NG, OUT), jnp.float32),
    )(psum, pcnt, clf_W, clf_b)


def kernel(x, edge_index, batch, W, att_src, att_dst, bias_gat, ln_gamma,
           ln_beta, clf_W, clf_b):
    x_pad = jnp.pad(x, ((0, NP - N), (0, 0)))
    att2 = jnp.stack([att_src, att_dst], axis=1)
    h, a2 = _pre(x_pad, W, att2)
    a_src = a2[:, 0]
    a_dst = a2[:, 1]
    pad = jnp.full((EP - E,), N, dtype=edge_index.dtype)
    srcp = jnp.concatenate([edge_index[0], pad])
    dstp = jnp.concatenate([edge_index[1], pad])
    epk = ((srcp << 14) | dstp).reshape(EP // CHUNK, CHUNK)
    accf, den = _edge_all(h, a_src, a_dst, epk)
    acc = accf.reshape(2, NHT, D)
    den = den.reshape(2, 16, NHT)
    batch3 = jnp.pad(batch, (0, NP - N), constant_values=NG).reshape(NBLK, 1, BLK)
    psum, pcnt = _post(acc, den, h, a2, x_pad, batch3, bias_gat.reshape(1, D),
                       ln_gamma.reshape(1, D), ln_beta.reshape(1, D))
    return _clf(psum, pcnt, clf_W, clf_b.reshape(1, OUT))


# speedup vs baseline: 10.7933x; 1.0013x over previous
import jax
import jax.numpy as jnp
from jax import lax
from jax.experimental import pallas as pl
from jax.experimental.pallas import tpu as pltpu
from jax.experimental.pallas import tpu_sc as plsc

N = 10000
NP = 10240
E = 320000
EP = 327680
D = 128
NG = 64
OUT = 323
NTILES = 32
CHUNK = 128
NH = NP // 2
NHT = NH + CHUNK
CHUNKS_PER_TILE = EP // 16 // CHUNK
ROWS_PER_TILE = NHT // 16
NBLK = 8
BLK = NP // NBLK


def _pre_body(x_ref, w_ref, att_ref, h_ref, a2_ref):
    h = jnp.dot(x_ref[...], w_ref[...], preferred_element_type=jnp.float32)
    h_ref[...] = h
    a2_ref[...] = jnp.dot(h, att_ref[...], preferred_element_type=jnp.float32)


def _pre(x_pad, W, att2):
    return pl.pallas_call(
        _pre_body,
        grid=(NBLK,),
        in_specs=[
            pl.BlockSpec((BLK, D), lambda i: (i, 0)),
            pl.BlockSpec((D, D), lambda i: (0, 0)),
            pl.BlockSpec((D, 2), lambda i: (0, 0)),
        ],
        out_specs=[
            pl.BlockSpec((BLK, D), lambda i: (i, 0)),
            pl.BlockSpec((BLK, 2), lambda i: (i, 0)),
        ],
        out_shape=[
            jax.ShapeDtypeStruct((NP, D), jnp.float32),
            jax.ShapeDtypeStruct((NP, 2), jnp.float32),
        ],
    )(x_pad, W, att2)


def _make_edge_body():
    def _edge_body(h_hbm, asrc_hbm, adst_hbm, epk_hbm, out_hbm,
                   den_hbm, asrc_v, adst_v, epk2d, rows_a, rows_b,
                   wbuf_a, wbuf_b, sidx_a, sidx_b, dloc_a, dloc_b,
                   den_v, acc, gsem_a, gsem_b, ssem_a, ssem_b):
        c = lax.axis_index("c")
        s = lax.axis_index("s")
        base_node = c * NH
        out_base = c * NHT

        pltpu.sync_copy(asrc_hbm, asrc_v)
        pltpu.sync_copy(adst_hbm, adst_v)
        pltpu.sync_copy(epk_hbm.at[pl.ds(s * CHUNKS_PER_TILE, CHUNKS_PER_TILE)],
                        epk2d)

        zero16 = jnp.zeros((16,), jnp.float32)

        def _zero_row(i, carry):
            for k in range(D // 16):
                rows_a[i, pl.ds(k * 16, 16)] = zero16
            return carry
        lax.fori_loop(0, CHUNK, _zero_row, 0)

        def _zero_den(i, carry):
            den_v[pl.ds(i * 16, 16)] = zero16
            return carry
        lax.fori_loop(0, NHT // 16, _zero_den, 0)

        pltpu.sync_copy(rows_a, acc.at[pl.ds(s * ROWS_PER_TILE, CHUNK)])
        pltpu.sync_copy(rows_a,
                        acc.at[pl.ds(s * ROWS_PER_TILE + CHUNK, CHUNK)])
        pltpu.sync_copy(rows_a.at[pl.ds(0, ROWS_PER_TILE - 2 * CHUNK)],
                        acc.at[pl.ds(s * ROWS_PER_TILE + 2 * CHUNK,
                                     ROWS_PER_TILE - 2 * CHUNK)])
        plsc.subcore_barrier()

        lanes = lax.iota(jnp.int32, 16)

        def _compute(g, sidx_b, dloc_b, wbuf_b):
            for j in range(CHUNK // 16):
                pk = epk2d[g, pl.ds(j * 16, 16)]
                s16 = lax.shift_right_logical(pk, 14)
                d16 = pk & ((1 << 14) - 1)
                sidx_b[pl.ds(j * 16, 16)] = s16
                dl = d16 - base_node
                dl = jnp.where((dl >= 0) & (dl < NH), dl, NH)
                dloc_b[pl.ds(j * 16, 16)] = dl
                e = (plsc.load_gather(asrc_v, [s16])
                     + plsc.load_gather(adst_v, [d16]))
                e = jnp.maximum(e, 0.2 * e)
                w16 = jnp.exp(e)
                wbuf_b[pl.ds(j * 16, 16)] = w16
                plsc.addupdate_scatter(den_v, [dl], w16)

        def _scale(rows_x, wbuf_x):
            def _grp(j, carry2):
                w16 = wbuf_x[pl.ds(j * 16, 16)]
                base = j * 16
                for l in range(16):
                    ws = w16[l]
                    for k in range(D // 16):
                        rows_x[base + l, pl.ds(k * 16, 16)] = (
                            rows_x[base + l, pl.ds(k * 16, 16)] * ws)
                return carry2
            lax.fori_loop(0, CHUNK // 16, _grp, 0)

        sets = ((sidx_a, dloc_a, wbuf_a, rows_a, gsem_a, ssem_a),
                (sidx_b, dloc_b, wbuf_b, rows_b, gsem_b, ssem_b))

        _compute(0, sidx_a, dloc_a, wbuf_a)
        pltpu.async_copy(h_hbm.at[sidx_a], rows_a, gsem_a)

        def _pair(t, carry):
            for half in range(2):
                sx, dx, wx, rx, gx, ssx = sets[1 - half]
                sy, dy, wy, ry, gy, ssy = sets[half]
                nxt = 2 * t + 1 + half
                cur = 2 * t + half

                @pl.when(nxt < CHUNKS_PER_TILE)
                def _fill():
                    @pl.when(nxt >= 2)
                    def _drain_prev_scatter():
                        pltpu.make_async_copy(rx, acc.at[dx], ssx).wait()
                    _compute(nxt, sx, dx, wx)
                    pltpu.async_copy(h_hbm.at[sx], rx, gx)

                pltpu.make_async_copy(h_hbm.at[sy], ry, gy).wait()
                _scale(ry, wy)
                pltpu.async_copy(ry, acc.at[dy], ssy, add=True)
            return carry
        lax.fori_loop(0, CHUNKS_PER_TILE // 2, _pair, 0)

        pltpu.make_async_copy(rows_a, acc.at[dloc_a], ssem_a).wait()
        pltpu.make_async_copy(rows_b, acc.at[dloc_b], ssem_b).wait()

        plsc.subcore_barrier()
        pltpu.sync_copy(acc.at[pl.ds(s * ROWS_PER_TILE, ROWS_PER_TILE)],
                        out_hbm.at[pl.ds(out_base + s * ROWS_PER_TILE,
                                         ROWS_PER_TILE)])
        pltpu.sync_copy(den_v, den_hbm.at[c * 16 + s])

    return _edge_body


def _edge_all(h, a_src, a_dst, epk):
    mesh = plsc.VectorSubcoreMesh(core_axis_name="c", subcore_axis_name="s",
                                  num_cores=2, num_subcores=16)
    fn = pl.kernel(
        _make_edge_body(),
        out_type=[
            jax.ShapeDtypeStruct((2 * NHT, D), jnp.float32),
            jax.ShapeDtypeStruct((NTILES, NHT), jnp.float32),
        ],
        mesh=mesh,
        scratch_types=[
            pltpu.VMEM((NP,), jnp.float32),
            pltpu.VMEM((NP,), jnp.float32),
            pltpu.VMEM((CHUNKS_PER_TILE, CHUNK), jnp.int32),
            pltpu.VMEM((CHUNK, D), jnp.float32),
            pltpu.VMEM((CHUNK, D), jnp.float32),
            pltpu.VMEM((CHUNK,), jnp.float32),
            pltpu.VMEM((CHUNK,), jnp.float32),
            pltpu.VMEM((CHUNK,), jnp.int32),
            pltpu.VMEM((CHUNK,), jnp.int32),
            pltpu.VMEM((CHUNK,), jnp.int32),
            pltpu.VMEM((CHUNK,), jnp.int32),
            pltpu.VMEM((NHT,), jnp.float32),
            pltpu.VMEM_SHARED((NHT, D), jnp.float32),
            pltpu.SemaphoreType.DMA,
            pltpu.SemaphoreType.DMA,
            pltpu.SemaphoreType.DMA,
            pltpu.SemaphoreType.DMA,
        ],
        compiler_params=pltpu.CompilerParams(needs_layout_passes=False),
    )
    return fn(h, a_src, a_dst, epk)


def _post_body(acc_ref, den_ref, h_ref, a2_ref, x_ref, batch_ref, bias_ref,
               g_ref, b_ref, psum_ref, pcnt_ref):
    i = pl.program_id(0)

    @pl.when(i == 0)
    def _init():
        psum_ref[...] = jnp.zeros_like(psum_ref)
        pcnt_ref[...] = jnp.zeros_like(pcnt_ref)

    rows = acc_ref[0]
    dsum = jnp.sum(den_ref[0], axis=0, keepdims=True)
    den = lax.transpose(dsum, (1, 0))
    a2b = a2_ref[...]
    es = a2b[:, 0:1] + a2b[:, 1:2]
    es = jnp.maximum(es, 0.2 * es)
    wself = jnp.exp(es)
    rows = rows + wself * h_ref[...]
    den = den + wself
    agg = rows / (den + 1e-16) + bias_ref[...]
    h2 = agg + x_ref[...]
    mu = jnp.mean(h2, axis=1, keepdims=True)
    var = jnp.mean((h2 - mu) ** 2, axis=1, keepdims=True)
    hn = (h2 - mu) / jnp.sqrt(var + 1e-5) * g_ref[...] + b_ref[...]

    batch = batch_ref[0, 0]
    gids = lax.broadcasted_iota(jnp.int32, (NG, BLK), 0)
    bb = jnp.broadcast_to(batch.reshape(1, BLK), (NG, BLK))
    rid = i * BLK + lax.broadcasted_iota(jnp.int32, (NG, BLK), 1)
    oh = jnp.where((bb == gids) & (rid < N), jnp.float32(1.0), jnp.float32(0.0))
    psum_ref[...] += jnp.dot(oh, hn, preferred_element_type=jnp.float32)
    pcnt_ref[...] += jnp.broadcast_to(
        jnp.sum(oh, axis=1, keepdims=True), (NG, D))


def _post(acc, den, h, a2, x_pad, batch3, bias, gamma, beta):
    return pl.pallas_call(
        _post_body,
        grid=(NBLK,),
        in_specs=[
            pl.BlockSpec((1, BLK, D), lambda i: (i // 4, i % 4, 0)),
            pl.BlockSpec((1, 16, BLK), lambda i: (i // 4, 0, i % 4)),
            pl.BlockSpec((BLK, D), lambda i: (i, 0)),
            pl.BlockSpec((BLK, 2), lambda i: (i, 0)),
            pl.BlockSpec((BLK, D), lambda i: (i, 0)),
            pl.BlockSpec((1, 1, BLK), lambda i: (i, 0, 0)),
            pl.BlockSpec((1, D), lambda i: (0, 0)),
            pl.BlockSpec((1, D), lambda i: (0, 0)),
            pl.BlockSpec((1, D), lambda i: (0, 0)),
        ],
        out_specs=[
            pl.BlockSpec((NG, D), lambda i: (0, 0)),
            pl.BlockSpec((NG, D), lambda i: (0, 0)),
        ],
        out_shape=[
            jax.ShapeDtypeStruct((NG, D), jnp.float32),
            jax.ShapeDtypeStruct((NG, D), jnp.float32),
        ],
    )(acc, den, h, a2, x_pad, batch3, bias, gamma, beta)


def _clf_body(psum_ref, pcnt_ref, w_ref, b_ref, out_ref):
    pooled = psum_ref[...] / jnp.maximum(pcnt_ref[...], 1.0)
    out_ref[...] = jnp.dot(pooled, w_ref[...],
                           preferred_element_type=jnp.float32) + b_ref[...]


def _clf(psum, pcnt, clf_W, clf_b):
    return pl.pallas_call(
        _clf_body,
        out_shape=jax.ShapeDtypeStruct((NG, OUT), jnp.float32),
    )(psum, pcnt, clf_W, clf_b)


def kernel(x, edge_index, batch, W, att_src, att_dst, bias_gat, ln_gamma,
           ln_beta, clf_W, clf_b):
    x_pad = jnp.pad(x, ((0, NP - N), (0, 0)))
    att2 = jnp.stack([att_src, att_dst], axis=1)
    h, a2 = _pre(x_pad, W, att2)
    a_src = a2[:, 0]
    a_dst = a2[:, 1]
    pad = jnp.full((EP - E,), N, dtype=edge_index.dtype)
    srcp = jnp.concatenate([edge_index[0], pad])
    dstp = jnp.concatenate([edge_index[1], pad])
    epk = ((srcp << 14) | dstp).reshape(EP // CHUNK, CHUNK)
    accf, den = _edge_all(h, a_src, a_dst, epk)
    acc = accf.reshape(2, NHT, D)
    den = den.reshape(2, 16, NHT)
    batch3 = jnp.pad(batch, (0, NP - N), constant_values=NG).reshape(NBLK, 1, BLK)
    psum, pcnt = _post(acc, den, h, a2, x_pad, batch3, bias_gat.reshape(1, D),
                       ln_gamma.reshape(1, D), ln_beta.reshape(1, D))
    return _clf(psum, pcnt, clf_W, clf_b.reshape(1, OUT))


# speedup vs baseline: 10.7982x; 1.0005x over previous
import jax
import jax.numpy as jnp
from jax import lax
from jax.experimental import pallas as pl
from jax.experimental.pallas import tpu as pltpu
from jax.experimental.pallas import tpu_sc as plsc

N = 10000
NP = 10240
E = 320000
EP = 327680
D = 128
NG = 64
OUT = 323
NTILES = 32
CHUNK = 128
NH = NP // 2
NHT = NH + CHUNK
CHUNKS_PER_TILE = EP // 16 // CHUNK
ROWS_PER_TILE = NHT // 16
NBLK = 8
BLK = NP // NBLK


def _pre_body(x_ref, w_ref, att_ref, h_ref, a2_ref):
    h = jnp.dot(x_ref[...], w_ref[...], preferred_element_type=jnp.float32)
    h_ref[...] = h
    a2_ref[...] = jnp.dot(h, att_ref[...], preferred_element_type=jnp.float32)


def _pre(x_pad, W, att2):
    return pl.pallas_call(
        _pre_body,
        grid=(NBLK,),
        in_specs=[
            pl.BlockSpec((BLK, D), lambda i: (i, 0)),
            pl.BlockSpec((D, D), lambda i: (0, 0)),
            pl.BlockSpec((D, 2), lambda i: (0, 0)),
        ],
        out_specs=[
            pl.BlockSpec((BLK, D), lambda i: (i, 0)),
            pl.BlockSpec((BLK, 2), lambda i: (i, 0)),
        ],
        out_shape=[
            jax.ShapeDtypeStruct((NP, D), jnp.float32),
            jax.ShapeDtypeStruct((NP, 2), jnp.float32),
        ],
    )(x_pad, W, att2)


def _make_edge_body():
    def _edge_body(h_hbm, asrc_hbm, adst_hbm, epk_hbm, out_hbm,
                   den_hbm, asrc_v, adst_v, epk2d, rows_a, rows_b,
                   wbuf_a, wbuf_b, sidx_a, sidx_b, dloc_a, dloc_b,
                   den_v, acc, gsem_a, gsem_b, ssem_a, ssem_b):
        c = lax.axis_index("c")
        s = lax.axis_index("s")
        base_node = c * NH
        out_base = c * NHT

        pltpu.sync_copy(asrc_hbm, asrc_v)
        pltpu.sync_copy(adst_hbm, adst_v)
        pltpu.sync_copy(epk_hbm.at[pl.ds(s * CHUNKS_PER_TILE, CHUNKS_PER_TILE)],
                        epk2d)

        zero16 = jnp.zeros((16,), jnp.float32)

        def _zero_row(i, carry):
            for k in range(D // 16):
                rows_a[i, pl.ds(k * 16, 16)] = zero16
            return carry
        lax.fori_loop(0, CHUNK, _zero_row, 0)

        def _zero_den(i, carry):
            den_v[pl.ds(i * 16, 16)] = zero16
            return carry
        lax.fori_loop(0, NHT // 16, _zero_den, 0)

        pltpu.sync_copy(rows_a, acc.at[pl.ds(s * ROWS_PER_TILE, CHUNK)])
        pltpu.sync_copy(rows_a,
                        acc.at[pl.ds(s * ROWS_PER_TILE + CHUNK, CHUNK)])
        pltpu.sync_copy(rows_a.at[pl.ds(0, ROWS_PER_TILE - 2 * CHUNK)],
                        acc.at[pl.ds(s * ROWS_PER_TILE + 2 * CHUNK,
                                     ROWS_PER_TILE - 2 * CHUNK)])
        plsc.subcore_barrier()

        def _compute(g, sidx_b, dloc_b, wbuf_b):
            for j in range(CHUNK // 16):
                pk = epk2d[g, pl.ds(j * 16, 16)]
                s16 = lax.shift_right_logical(pk, 14)
                d16 = pk & ((1 << 14) - 1)
                sidx_b[pl.ds(j * 16, 16)] = s16
                dl = d16 - base_node
                dl = jnp.where((dl >= 0) & (dl < NH), dl, NH)
                dloc_b[pl.ds(j * 16, 16)] = dl
                e = (plsc.load_gather(asrc_v, [s16])
                     + plsc.load_gather(adst_v, [d16]))
                e = jnp.maximum(e, 0.2 * e)
                w16 = jnp.exp(e)
                wbuf_b[pl.ds(j * 16, 16)] = w16
                plsc.addupdate_scatter(den_v, [dl], w16)

        def _scale(rows_x, wbuf_x):
            def _grp(j, carry2):
                w16 = wbuf_x[pl.ds(j * 16, 16)]
                base = j * 16
                for l in range(16):
                    ws = w16[l]
                    for k in range(D // 16):
                        rows_x[base + l, pl.ds(k * 16, 16)] = (
                            rows_x[base + l, pl.ds(k * 16, 16)] * ws)
                return carry2
            lax.fori_loop(0, CHUNK // 16, _grp, 0)

        sets = ((sidx_a, dloc_a, wbuf_a, rows_a, gsem_a, ssem_a),
                (sidx_b, dloc_b, wbuf_b, rows_b, gsem_b, ssem_b))

        _compute(0, sidx_a, dloc_a, wbuf_a)
        pltpu.async_copy(h_hbm.at[sidx_a], rows_a, gsem_a)

        def _pair(t, carry):
            for half in range(2):
                sx, dx, wx, rx, gx, ssx = sets[1 - half]
                sy, dy, wy, ry, gy, ssy = sets[half]
                nxt = 2 * t + 1 + half
                cur = 2 * t + half

                @pl.when(nxt < CHUNKS_PER_TILE)
                def _fill():
                    @pl.when(nxt >= 2)
                    def _drain_prev_scatter():
                        pltpu.make_async_copy(rx, acc.at[dx], ssx).wait()
                    _compute(nxt, sx, dx, wx)
                    pltpu.async_copy(h_hbm.at[sx], rx, gx)

                pltpu.make_async_copy(h_hbm.at[sy], ry, gy).wait()
                _scale(ry, wy)
                pltpu.async_copy(ry, acc.at[dy], ssy, add=True)
            return carry
        lax.fori_loop(0, CHUNKS_PER_TILE // 2, _pair, 0)

        pltpu.make_async_copy(rows_a, acc.at[dloc_a], ssem_a).wait()
        pltpu.make_async_copy(rows_b, acc.at[dloc_b], ssem_b).wait()

        plsc.subcore_barrier()
        pltpu.sync_copy(acc.at[pl.ds(s * ROWS_PER_TILE, ROWS_PER_TILE)],
                        out_hbm.at[pl.ds(out_base + s * ROWS_PER_TILE,
                                         ROWS_PER_TILE)])
        pltpu.sync_copy(den_v, den_hbm.at[c * 16 + s])

    return _edge_body


def _edge_all(h, a_src, a_dst, epk):
    mesh = plsc.VectorSubcoreMesh(core_axis_name="c", subcore_axis_name="s",
                                  num_cores=2, num_subcores=16)
    fn = pl.kernel(
        _make_edge_body(),
        out_type=[
            jax.ShapeDtypeStruct((2 * NHT, D), jnp.float32),
            jax.ShapeDtypeStruct((NTILES, NHT), jnp.float32),
        ],
        mesh=mesh,
        scratch_types=[
            pltpu.VMEM((NP,), jnp.float32),
            pltpu.VMEM((NP,), jnp.float32),
            pltpu.VMEM((CHUNKS_PER_TILE, CHUNK), jnp.int32),
            pltpu.VMEM((CHUNK, D), jnp.float32),
            pltpu.VMEM((CHUNK, D), jnp.float32),
            pltpu.VMEM((CHUNK,), jnp.float32),
            pltpu.VMEM((CHUNK,), jnp.float32),
            pltpu.VMEM((CHUNK,), jnp.int32),
            pltpu.VMEM((CHUNK,), jnp.int32),
            pltpu.VMEM((CHUNK,), jnp.int32),
            pltpu.VMEM((CHUNK,), jnp.int32),
            pltpu.VMEM((NHT,), jnp.float32),
            pltpu.VMEM_SHARED((NHT, D), jnp.float32),
            pltpu.SemaphoreType.DMA,
            pltpu.SemaphoreType.DMA,
            pltpu.SemaphoreType.DMA,
            pltpu.SemaphoreType.DMA,
        ],
        compiler_params=pltpu.CompilerParams(needs_layout_passes=False),
    )
    return fn(h, a_src, a_dst, epk)


def _post_body(acc_ref, den_ref, h_ref, a2_ref, x_ref, batch_ref, bias_ref,
               g_ref, b_ref, psum_ref, pcnt_ref):
    i = pl.program_id(0)

    @pl.when(i == 0)
    def _init():
        psum_ref[...] = jnp.zeros_like(psum_ref)
        pcnt_ref[...] = jnp.zeros_like(pcnt_ref)

    rows = acc_ref[0]
    dsum = jnp.sum(den_ref[0], axis=0, keepdims=True)
    den = lax.transpose(dsum, (1, 0))
    a2b = a2_ref[...]
    es = a2b[:, 0:1] + a2b[:, 1:2]
    es = jnp.maximum(es, 0.2 * es)
    wself = jnp.exp(es)
    rows = rows + wself * h_ref[...]
    den = den + wself
    agg = rows / (den + 1e-16) + bias_ref[...]
    h2 = agg + x_ref[...]
    mu = jnp.mean(h2, axis=1, keepdims=True)
    var = jnp.mean((h2 - mu) ** 2, axis=1, keepdims=True)
    hn = (h2 - mu) / jnp.sqrt(var + 1e-5) * g_ref[...] + b_ref[...]

    batch = batch_ref[0, 0]
    gids = lax.broadcasted_iota(jnp.int32, (NG, BLK), 0)
    bb = jnp.broadcast_to(batch.reshape(1, BLK), (NG, BLK))
    rid = i * BLK + lax.broadcasted_iota(jnp.int32, (NG, BLK), 1)
    oh = jnp.where((bb == gids) & (rid < N), jnp.float32(1.0), jnp.float32(0.0))
    psum_ref[...] += jnp.dot(oh, hn, preferred_element_type=jnp.float32)
    pcnt_ref[...] += jnp.broadcast_to(
        jnp.sum(oh, axis=1, keepdims=True), (NG, D))


def _post(acc, den, h, a2, x_pad, batch3, bias, gamma, beta):
    return pl.pallas_call(
        _post_body,
        grid=(NBLK,),
        in_specs=[
            pl.BlockSpec((1, BLK, D), lambda i: (i // 4, i % 4, 0)),
            pl.BlockSpec((1, 16, BLK), lambda i: (i // 4, 0, i % 4)),
            pl.BlockSpec((BLK, D), lambda i: (i, 0)),
            pl.BlockSpec((BLK, 2), lambda i: (i, 0)),
            pl.BlockSpec((BLK, D), lambda i: (i, 0)),
            pl.BlockSpec((1, 1, BLK), lambda i: (i, 0, 0)),
            pl.BlockSpec((1, D), lambda i: (0, 0)),
            pl.BlockSpec((1, D), lambda i: (0, 0)),
            pl.BlockSpec((1, D), lambda i: (0, 0)),
        ],
        out_specs=[
            pl.BlockSpec((NG, D), lambda i: (0, 0)),
            pl.BlockSpec((NG, D), lambda i: (0, 0)),
        ],
        out_shape=[
            jax.ShapeDtypeStruct((NG, D), jnp.float32),
            jax.ShapeDtypeStruct((NG, D), jnp.float32),
        ],
    )(acc, den, h, a2, x_pad, batch3, bias, gamma, beta)


def _clf_body(psum_ref, pcnt_ref, w_ref, b_ref, out_ref):
    pooled = psum_ref[...] / jnp.maximum(pcnt_ref[...], 1.0)
    out_ref[...] = jnp.dot(pooled, w_ref[...],
                           preferred_element_type=jnp.float32) + b_ref[...]


def _clf(psum, pcnt, clf_W, clf_b):
    return pl.pallas_call(
        _clf_body,
        out_shape=jax.ShapeDtypeStruct((NG, OUT), jnp.float32),
    )(psum, pcnt, clf_W, clf_b)


def kernel(x, edge_index, batch, W, att_src, att_dst, bias_gat, ln_gamma,
           ln_beta, clf_W, clf_b):
    x_pad = jnp.pad(x, ((0, NP - N), (0, 0)))
    att2 = jnp.stack([att_src, att_dst], axis=1)
    h, a2 = _pre(x_pad, W, att2)
    a_src = a2[:, 0]
    a_dst = a2[:, 1]
    pad = jnp.full((EP - E,), N, dtype=edge_index.dtype)
    srcp = jnp.concatenate([edge_index[0], pad])
    dstp = jnp.concatenate([edge_index[1], pad])
    epk = ((srcp << 14) | dstp).reshape(EP // CHUNK, CHUNK)
    accf, den = _edge_all(h, a_src, a_dst, epk)
    acc = accf.reshape(2, NHT, D)
    den = den.reshape(2, 16, NHT)
    batch3 = jnp.pad(batch, (0, NP - N), constant_values=NG).reshape(NBLK, 1, BLK)
    psum, pcnt = _post(acc, den, h, a2, x_pad, batch3, bias_gat.reshape(1, D),
                       ln_gamma.reshape(1, D), ln_beta.reshape(1, D))
    return _clf(psum, pcnt, clf_W, clf_b.reshape(1, OUT))
